# baseline (speedup 1.0000x reference)
import functools

import jax
import jax.numpy as jnp
from jax import lax
from jax.experimental import pallas as pl
from jax.experimental.pallas import tpu as pltpu
from jax.experimental.pallas import tpu_sc as plsc

_NF = 26
_E = 16
_V = 100000
_BATCH = 16384
_B = _BATCH * _NF
_NC, _NS = 2, 16
_NW = _NC * _NS
_TCOLS = 20313
_TPW = (_TCOLS + _NW - 1) // _NW
_SROWS = _TCOLS * 16
_BBLK = 128
_XBLK_PER_W = _BATCH // _BBLK // _NW
_IPB = _NF * _BBLK
_BPW = _B // _NW
_IDX_ROW = 128
_ROWS_PER_W = _BPW // _IDX_ROW
_CROWS = 13
_NCHUNK = _ROWS_PER_W // _CROWS
_CIDX = _CROWS * _IDX_ROW


def _k1_relayout(xT, tableT):
    mesh = plsc.VectorSubcoreMesh(core_axis_name="c", subcore_axis_name="s")

    @functools.partial(
        pl.kernel,
        out_type=(
            jax.ShapeDtypeStruct((_SROWS, 128), jnp.float32),
            jax.ShapeDtypeStruct((_B // 128, 128), jnp.int32),
        ),
        mesh=mesh,
        compiler_params=pltpu.CompilerParams(needs_layout_passes=False),
        scratch_types=[
            pltpu.VMEM((16, 128), jnp.float32),
            pltpu.VMEM((16, 128), jnp.float32),
            pltpu.VMEM((_NF, _BATCH // _NW), jnp.int32),
            pltpu.VMEM((_BPW // 128, 128), jnp.int32),
        ],
    )
    def k(xT_hbm, tT_hbm, srows_hbm, idx_hbm, tbuf, obuf, xbuf, ibuf):
        cid = lax.axis_index("c")
        sid = lax.axis_index("s")
        wid = sid * _NC + cid
        ib = lax.iota(jnp.int32, 16)

        def tcol_body(j, carry):
            t = wid + j * _NW

            @pl.when(t < _TCOLS)
            def _():
                c0 = pl.multiple_of(t * 128, 128)
                pltpu.sync_copy(tT_hbm.at[:, pl.ds(c0, 128)], tbuf)
                for c in range(128):
                    col = plsc.load_gather(tbuf, [ib, jnp.full((16,), c, jnp.int32)])
                    obuf[c // 8, pl.ds(16 * (c % 8), 16)] = col
                r0 = pl.multiple_of(t * 16, 16)
                pltpu.sync_copy(obuf, srows_hbm.at[pl.ds(r0, 16)])
            return carry

        lax.fori_loop(0, _TPW, tcol_body, 0)

        bpw = _BATCH // _NW
        b0 = pl.multiple_of(wid * bpw, 128)
        pltpu.sync_copy(xT_hbm.at[:, pl.ds(b0, bpw)], xbuf)

        def vreg_body(d, carry2):
            p = ib + d * 16
            f = p % _NF
            bl = p // _NF
            vals = plsc.load_gather(xbuf, [f, bl]) + f * _V
            row = d // 8
            colq = (d % 8) * 16
            ibuf[row, pl.ds(colq, 16)] = vals
            return carry2

        lax.fori_loop(0, _BPW // 16, vreg_body, 0)
        pltpu.sync_copy(ibuf, idx_hbm.at[pl.ds(wid * (_BPW // 128), _BPW // 128)])

    return k(xT, tableT)


def _k2_gather(table2, idx2):
    mesh = plsc.VectorSubcoreMesh(core_axis_name="c", subcore_axis_name="s")

    @functools.partial(
        pl.kernel,
        out_type=jax.ShapeDtypeStruct((_B, _E), jnp.float32),
        mesh=mesh,
        compiler_params=pltpu.CompilerParams(use_tc_tiling_on_sc=False),
        scratch_types=[
            pltpu.VMEM((_ROWS_PER_W, _IDX_ROW), jnp.int32),
            pltpu.VMEM((_CIDX, _E), jnp.float32),
            pltpu.SemaphoreType.DMA,
        ],
    )
    def k(idx_hbm, table_hbm, out_hbm, idx_v, rows_v, gsem):
        wid = lax.axis_index("s") * _NC + lax.axis_index("c")
        rbase = wid * _ROWS_PER_W
        obase = wid * _BPW
        pltpu.sync_copy(idx_hbm.at[pl.ds(rbase, _ROWS_PER_W)], idx_v)

        def chunk(c, carry):
            r0 = c * _CROWS
            descs = [
                pltpu.async_copy(
                    table_hbm.at[idx_v.at[r0 + t]],
                    rows_v.at[pl.ds(t * _IDX_ROW, _IDX_ROW)],
                    gsem,
                )
                for t in range(_CROWS)
            ]
            for d in descs:
                d.wait()
            pltpu.sync_copy(rows_v, out_hbm.at[pl.ds(obase + c * _CIDX, _CIDX)])
            return carry

        lax.fori_loop(0, _NCHUNK, chunk, 0)

    return k(idx2, table2)


def kernel(x, table):
    xT = x.T.astype(jnp.int32)
    tT = table.T
    srows, idx2 = _k1_relayout(xT, tT)
    table2 = srows.reshape(_SROWS * 8, _E)
    out = _k2_gather(table2, idx2)
    return out.reshape(_BATCH, _NF, _E)
